# baseline (speedup 1.0000x reference)
import functools
import jax
import jax.numpy as jnp
from jax import lax
from jax.experimental import pallas as pl
from jax.experimental.pallas import tpu as pltpu
from jax.experimental.pallas import tpu_sc as plsc

NC, NS, L = 2, 16, 16
NW = NC * NS
B = 16384
D = 64
DC = 128
BPW = B // NW
CH = 128
NCHUNK = BPW // CH

_mesh = plsc.VectorSubcoreMesh(core_axis_name="c", subcore_axis_name="s")


@functools.partial(
    pl.kernel,
    out_type=jax.ShapeDtypeStruct((B,), jnp.float32),
    mesh=_mesh,
    compiler_params=pltpu.CompilerParams(
        needs_layout_passes=False, use_tc_tiling_on_sc=False),
    scratch_types=[
        pltpu.VMEM((NCHUNK, CH), jnp.int32),
        pltpu.VMEM((NCHUNK, CH), jnp.int32),
        pltpu.VMEM((CH, DC // 2), jnp.int32),
        pltpu.VMEM((CH, DC // 2), jnp.int32),
        pltpu.VMEM((CH, DC // 2), jnp.int32),
        pltpu.VMEM((CH, DC // 2), jnp.int32),
        pltpu.VMEM((CH, L), jnp.float32),
        pltpu.VMEM((BPW,), jnp.float32),
        pltpu.SemaphoreType.DMA,
        pltpu.SemaphoreType.DMA,
    ],
)
def _two_tower(u_hbm, p_hbm, c_hbm, out_hbm,
               u_idx, p_idx,
               u_buf0, u_buf1, p_buf0, p_buf1,
               m_v, out_v, sem0, sem1):
    wid = lax.axis_index("s") * NC + lax.axis_index("c")
    base = wid * BPW
    ubufs = (u_buf0, u_buf1)
    pbufs = (p_buf0, p_buf1)
    sems = (sem0, sem1)

    for c in range(NCHUNK):
        pltpu.sync_copy(u_hbm.at[pl.ds(base + c * CH, CH)], u_idx.at[c])
        pltpu.sync_copy(p_hbm.at[pl.ds(base + c * CH, CH)], p_idx.at[c])

    def fire(c):
        bb = c % 2
        du = pltpu.async_copy(c_hbm.at[u_idx.at[c]], ubufs[bb], sems[bb])
        dp = pltpu.async_copy(c_hbm.at[p_idx.at[c]], pbufs[bb], sems[bb])
        return du, dp

    lanes = lax.iota(jnp.int32, L)

    def compute(c):
        bb = c % 2
        ub, pb = ubufs[bb], pbufs[bb]

        def item(i, carry):
            acc = jnp.zeros((L,), jnp.float32)
            for h in (0, L):
                uu = plsc.bitcast(ub[i, pl.ds(h, L)], jnp.bfloat16)
                pp = plsc.bitcast(pb[i, pl.ds(D // 2 + h, L)], jnp.bfloat16)
                ue, uo = plsc.unpack(uu, format=plsc.PackFormat.INTERLEAVED)
                pe, po = plsc.unpack(pp, format=plsc.PackFormat.INTERLEAVED)
                acc = acc + ue * pe + uo * po
            m_v[i, pl.ds(0, L)] = acc
            return carry

        lax.fori_loop(0, CH, item, 0)

        def redgroup(g, carry):
            rows = lanes + g * L
            tot = jnp.zeros((L,), jnp.float32)
            for k in range(L):
                col = jnp.full((L,), k, jnp.int32)
                tot = tot + plsc.load_gather(m_v, [rows, col])
            out_v[pl.ds(c * CH + g * L, L)] = 1.0 / (1.0 + jnp.exp(-tot))
            return carry

        lax.fori_loop(0, CH // L, redgroup, 0)

    pending = [fire(0), fire(1)]
    for c in range(NCHUNK):
        du, dp = pending[c]
        du.wait()
        dp.wait()
        compute(c)
        if c + 2 < NCHUNK:
            pending.append(fire(c + 2))

    pltpu.sync_copy(out_v, out_hbm.at[pl.ds(base, BPW)])


def kernel(u, p, user_table, prod_table):
    combined = jnp.concatenate(
        [user_table.astype(jnp.bfloat16), prod_table.astype(jnp.bfloat16)],
        axis=1)
    packed = jax.lax.bitcast_convert_type(
        combined.reshape(combined.shape[0], DC // 2, 2), jnp.int32)
    return _two_tower(u, p, packed)


# speedup vs baseline: 4.0489x; 4.0489x over previous
import functools
import jax
import jax.numpy as jnp
from jax import lax
from jax.experimental import pallas as pl
from jax.experimental.pallas import tpu as pltpu
from jax.experimental.pallas import tpu_sc as plsc

NC, NS, L = 2, 16, 16
NW = NC * NS
B = 16384
D = 64
N = 1000000
W = 512
NBLK = 977
S = NBLK * W
BPW = B // NW
CH = 128
NCHUNK = BPW // CH


def _pack_words(bits):
    lo = bits[0:32, :].astype(jnp.uint32)
    hi = bits[32:64, :].astype(jnp.uint32)
    return lo | (hi << 16)


def _relayout_body(au_ref, ap_ref, bu_ref, bp_ref, out_ref):
    def words(ref):
        bits = lax.bitcast_convert_type(
            ref[...].astype(jnp.bfloat16), jnp.uint16)
        return _pack_words(bits)

    ma = jnp.concatenate([words(au_ref), words(ap_ref)], axis=0)
    mb = jnp.concatenate([words(bu_ref), words(bp_ref)], axis=0)
    out = jnp.concatenate([ma.T, mb.T], axis=1)
    out_ref[...] = lax.bitcast_convert_type(out, jnp.int32)


_relayout = pl.pallas_call(
    _relayout_body,
    grid=(NBLK,),
    in_specs=[
        pl.BlockSpec((D, W), lambda b: (0, b)),
        pl.BlockSpec((D, W), lambda b: (0, b)),
        pl.BlockSpec((D, W), lambda b: (0, b + NBLK)),
        pl.BlockSpec((D, W), lambda b: (0, b + NBLK)),
    ],
    out_specs=pl.BlockSpec((W, 2 * D), lambda b: (b, 0)),
    out_shape=jax.ShapeDtypeStruct((S, 2 * D), jnp.int32),
)

_mesh = plsc.VectorSubcoreMesh(core_axis_name="c", subcore_axis_name="s")


@functools.partial(
    pl.kernel,
    out_type=jax.ShapeDtypeStruct((B,), jnp.float32),
    mesh=_mesh,
    compiler_params=pltpu.CompilerParams(
        needs_layout_passes=False, use_tc_tiling_on_sc=True),
    scratch_types=[
        pltpu.VMEM((NCHUNK, CH), jnp.int32),
        pltpu.VMEM((NCHUNK, CH), jnp.int32),
        pltpu.VMEM((NCHUNK, CH), jnp.int32),
        pltpu.VMEM((NCHUNK, CH), jnp.int32),
        pltpu.VMEM((CH, 2 * D), jnp.int32),
        pltpu.VMEM((CH, 2 * D), jnp.int32),
        pltpu.VMEM((CH, 2 * D), jnp.int32),
        pltpu.VMEM((CH, 2 * D), jnp.int32),
        pltpu.VMEM((CH, L), jnp.float32),
        pltpu.VMEM((BPW,), jnp.float32),
        pltpu.SemaphoreType.DMA,
        pltpu.SemaphoreType.DMA,
    ],
)
def _two_tower(u_hbm, p_hbm, g_hbm, out_hbm,
               u_idx, p_idx, u_row, p_row,
               u_buf0, u_buf1, p_buf0, p_buf1,
               m_v, out_v, sem0, sem1):
    wid = lax.axis_index("s") * NC + lax.axis_index("c")
    base = wid * BPW
    ubufs = (u_buf0, u_buf1)
    pbufs = (p_buf0, p_buf1)
    sems = (sem0, sem1)

    for c in range(NCHUNK):
        pltpu.sync_copy(u_hbm.at[pl.ds(base + c * CH, CH)], u_idx.at[c])
        pltpu.sync_copy(p_hbm.at[pl.ds(base + c * CH, CH)], p_idx.at[c])

    for c in range(NCHUNK):
        def torow(j, carry, c=c):
            uv = u_idx[c, pl.ds(j * L, L)]
            pv = p_idx[c, pl.ds(j * L, L)]
            u_row[c, pl.ds(j * L, L)] = jnp.where(uv >= S, uv - S, uv)
            p_row[c, pl.ds(j * L, L)] = jnp.where(pv >= S, pv - S, pv)
            return carry
        lax.fori_loop(0, CH // L, torow, 0)

    def fire(c):
        bb = c % 2
        du = pltpu.async_copy(g_hbm.at[u_row.at[c]], ubufs[bb], sems[bb])
        dp = pltpu.async_copy(g_hbm.at[p_row.at[c]], pbufs[bb], sems[bb])
        return du, dp

    lanes = lax.iota(jnp.int32, L)

    def compute(c):
        bb = c % 2
        ub, pb = ubufs[bb], pbufs[bb]

        def itemgroup(g, carry):
            uvec = u_idx[c, pl.ds(g * L, L)]
            pvec = p_idx[c, pl.ds(g * L, L)]
            uoffv = jnp.where(uvec >= S, 2 * D // 2, 0)
            poffv = jnp.where(pvec >= S, 2 * D // 2, 0)
            for j in range(L):
                i = g * L + j
                uoff = uoffv[j]
                poff = poffv[j]
                acc = jnp.zeros((L,), jnp.float32)
                for h in (0, L):
                    uu = plsc.bitcast(ub[i, pl.ds(uoff + h, L)],
                                      jnp.bfloat16)
                    pp = plsc.bitcast(pb[i, pl.ds(poff + 2 * L + h, L)],
                                      jnp.bfloat16)
                    ue, uo = plsc.unpack(
                        uu, format=plsc.PackFormat.INTERLEAVED)
                    pe, po = plsc.unpack(
                        pp, format=plsc.PackFormat.INTERLEAVED)
                    acc = acc + ue * pe + uo * po
                m_v[i, pl.ds(0, L)] = acc
            return carry

        lax.fori_loop(0, CH // L, itemgroup, 0)

        def redgroup(g, carry):
            rows = lanes + g * L
            tot = jnp.zeros((L,), jnp.float32)
            for k in range(L):
                col = jnp.full((L,), k, jnp.int32)
                tot = tot + plsc.load_gather(m_v, [rows, col])
            out_v[pl.ds(c * CH + g * L, L)] = 1.0 / (1.0 + jnp.exp(-tot))
            return carry

        lax.fori_loop(0, CH // L, redgroup, 0)

    pending = [fire(0), fire(1)]
    for c in range(NCHUNK):
        du, dp = pending[c]
        du.wait()
        dp.wait()
        compute(c)
        if c + 2 < NCHUNK:
            pending.append(fire(c + 2))

    pltpu.sync_copy(out_v, out_hbm.at[pl.ds(base, BPW)])


def kernel(u, p, user_table, prod_table):
    g = _relayout(user_table.T, prod_table.T, user_table.T, prod_table.T)
    tail_q = N - W * (2 * NBLK - 1)
    tu = lax.bitcast_convert_type(
        user_table[N - tail_q:, :].astype(jnp.bfloat16), jnp.uint16)
    tp = lax.bitcast_convert_type(
        prod_table[N - tail_q:, :].astype(jnp.bfloat16), jnp.uint16)

    def rowpack(bits):
        lo = bits[:, 0:32].astype(jnp.uint32)
        hi = bits[:, 32:64].astype(jnp.uint32)
        return lax.bitcast_convert_type(lo | (hi << 16), jnp.int32)

    patch = jnp.concatenate([rowpack(tu), rowpack(tp)], axis=1)
    g = lax.dynamic_update_slice(g, patch, (N - tail_q - S, 2 * D // 2))
    return _two_tower(u, p, g)


# speedup vs baseline: 4.0511x; 1.0005x over previous
import functools
import jax
import jax.numpy as jnp
from jax import lax
from jax.experimental import pallas as pl
from jax.experimental.pallas import tpu as pltpu
from jax.experimental.pallas import tpu_sc as plsc

NC, NS, L = 2, 16, 16
NW = NC * NS
B = 16384
D = 64
N = 1000000
W = 512
NBLK = 977
S = NBLK * W
BPW = B // NW
CH = 128
NCHUNK = BPW // CH


def _pack_words(bits):
    lo = bits[0:32, :].astype(jnp.uint32)
    hi = bits[32:64, :].astype(jnp.uint32)
    return lo | (hi << 16)


def _relayout_body(au_ref, ap_ref, bu_ref, bp_ref, out_ref):
    def words(ref):
        bits = lax.bitcast_convert_type(
            ref[...].astype(jnp.bfloat16), jnp.uint16)
        return _pack_words(bits)

    ma = jnp.concatenate([words(au_ref), words(ap_ref)], axis=0)
    mb = jnp.concatenate([words(bu_ref), words(bp_ref)], axis=0)
    out = jnp.concatenate([ma.T, mb.T], axis=1)
    out_ref[...] = lax.bitcast_convert_type(out, jnp.int32)


_relayout = pl.pallas_call(
    _relayout_body,
    grid=(NBLK,),
    in_specs=[
        pl.BlockSpec((D, W), lambda b: (0, b)),
        pl.BlockSpec((D, W), lambda b: (0, b)),
        pl.BlockSpec((D, W), lambda b: (0, b + NBLK)),
        pl.BlockSpec((D, W), lambda b: (0, b + NBLK)),
    ],
    out_specs=pl.BlockSpec((W, 2 * D), lambda b: (b, 0)),
    out_shape=jax.ShapeDtypeStruct((S, 2 * D), jnp.int32),
)

_mesh = plsc.VectorSubcoreMesh(core_axis_name="c", subcore_axis_name="s")


@functools.partial(
    pl.kernel,
    out_type=jax.ShapeDtypeStruct((B,), jnp.float32),
    mesh=_mesh,
    compiler_params=pltpu.CompilerParams(
        needs_layout_passes=False, use_tc_tiling_on_sc=True),
    scratch_types=[
        pltpu.VMEM((NCHUNK, CH), jnp.int32),
        pltpu.VMEM((NCHUNK, CH), jnp.int32),
        pltpu.VMEM((NCHUNK, CH), jnp.int32),
        pltpu.VMEM((NCHUNK, CH), jnp.int32),
        pltpu.VMEM((CH, 2 * D), jnp.int32),
        pltpu.VMEM((CH, 2 * D), jnp.int32),
        pltpu.VMEM((CH, 2 * D), jnp.int32),
        pltpu.VMEM((CH, 2 * D), jnp.int32),
        pltpu.VMEM((CH, L), jnp.float32),
        pltpu.VMEM((BPW,), jnp.float32),
        pltpu.SemaphoreType.DMA,
        pltpu.SemaphoreType.DMA,
    ],
)
def _two_tower(u_hbm, p_hbm, g_hbm, out_hbm,
               u_idx, p_idx, u_row, p_row,
               u_buf0, u_buf1, p_buf0, p_buf1,
               m_v, out_v, sem0, sem1):
    wid = lax.axis_index("s") * NC + lax.axis_index("c")
    base = wid * BPW
    ubufs = (u_buf0, u_buf1)
    pbufs = (p_buf0, p_buf1)
    sems = (sem0, sem1)

    for c in range(NCHUNK):
        pltpu.sync_copy(u_hbm.at[pl.ds(base + c * CH, CH)], u_idx.at[c])
        pltpu.sync_copy(p_hbm.at[pl.ds(base + c * CH, CH)], p_idx.at[c])

    for c in range(NCHUNK):
        def torow(j, carry, c=c):
            uv = u_idx[c, pl.ds(j * L, L)]
            pv = p_idx[c, pl.ds(j * L, L)]
            u_row[c, pl.ds(j * L, L)] = jnp.where(uv >= S, uv - S, uv)
            p_row[c, pl.ds(j * L, L)] = jnp.where(pv >= S, pv - S, pv)
            return carry
        lax.fori_loop(0, CH // L, torow, 0)

    def fire(c):
        bb = c % 2
        du = pltpu.async_copy(g_hbm.at[u_row.at[c]], ubufs[bb], sems[bb])
        dp = pltpu.async_copy(g_hbm.at[p_row.at[c]], pbufs[bb], sems[bb])
        return du, dp

    lanes = lax.iota(jnp.int32, L)

    def compute(c):
        bb = c % 2
        ub, pb = ubufs[bb], pbufs[bb]

        def itemgroup(g, carry):
            uvec = u_idx[c, pl.ds(g * L, L)]
            pvec = p_idx[c, pl.ds(g * L, L)]
            uoffv = jnp.where(uvec >= S, 2 * D // 2, 0)
            poffv = jnp.where(pvec >= S, 2 * D // 2, 0)
            for j in range(L):
                i = g * L + j
                uoff = uoffv[j]
                poff = poffv[j]
                acc = jnp.zeros((L,), jnp.float32)
                for h in (0, L):
                    uu = plsc.bitcast(ub[i, pl.ds(uoff + h, L)],
                                      jnp.bfloat16)
                    pp = plsc.bitcast(pb[i, pl.ds(poff + 2 * L + h, L)],
                                      jnp.bfloat16)
                    ue, uo = plsc.unpack(
                        uu, format=plsc.PackFormat.INTERLEAVED)
                    pe, po = plsc.unpack(
                        pp, format=plsc.PackFormat.INTERLEAVED)
                    acc = acc + ue * pe + uo * po
                m_v[i, pl.ds(0, L)] = acc
            return carry

        lax.fori_loop(0, CH // L, itemgroup, 0)

        def redgroup(g, carry):
            rows = lanes + g * L
            tot = jnp.zeros((L,), jnp.float32)
            for k in range(L):
                col = jnp.full((L,), k, jnp.int32)
                tot = tot + plsc.load_gather(m_v, [rows, col])
            out_v[pl.ds(c * CH + g * L, L)] = 1.0 / (1.0 + jnp.exp(-tot))
            return carry

        lax.fori_loop(0, CH // L, redgroup, 0)

    pending = [fire(0), fire(1)]
    for c in range(NCHUNK):
        du, dp = pending[c]
        du.wait()
        dp.wait()
        compute(c)
        if c + 2 < NCHUNK:
            pending.append(fire(c + 2))

    pltpu.sync_copy(out_v, out_hbm.at[pl.ds(base, BPW)])


def kernel(u, p, user_table, prod_table):
    g = _relayout(user_table.T, prod_table.T, user_table.T, prod_table.T)
    tail_q = N % W
    tu = lax.bitcast_convert_type(
        user_table[N - tail_q:, :].astype(jnp.bfloat16), jnp.uint16)
    tp = lax.bitcast_convert_type(
        prod_table[N - tail_q:, :].astype(jnp.bfloat16), jnp.uint16)

    def rowpack(bits):
        lo = bits[:, 0:32].astype(jnp.uint32)
        hi = bits[:, 32:64].astype(jnp.uint32)
        return lax.bitcast_convert_type(lo | (hi << 16), jnp.int32)

    patch = jnp.concatenate([rowpack(tu), rowpack(tp)], axis=1)
    g = lax.dynamic_update_slice(g, patch, (N - tail_q - S, 2 * D // 2))
    return _two_tower(u, p, g)
